# baseline (speedup 1.0000x reference)
import jax
import jax.numpy as jnp
from jax import lax
from jax.experimental import pallas as pl
from jax.experimental.pallas import tpu as pltpu
from jax.experimental.pallas import tpu_sc as plsc

_NUM_CORES = 2
_NUM_SUBCORES = 16
_NW = _NUM_CORES * _NUM_SUBCORES
_LANES = 16
_PACK = 4


def _sc_gather_packed(idx32, table2):
    batch = idx32.shape[0]
    width = table2.shape[1]
    b_per_w = batch // _NW

    mesh = plsc.VectorSubcoreMesh(core_axis_name="c", subcore_axis_name="s")

    @pl.kernel(
        out_type=jax.ShapeDtypeStruct((batch, width), table2.dtype),
        mesh=mesh,
        scratch_types=[
            pltpu.VMEM((b_per_w,), jnp.int32),
            pltpu.VMEM((b_per_w,), jnp.int32),
            pltpu.VMEM((b_per_w, width), table2.dtype),
            pltpu.SemaphoreType.DMA,
        ],
    )
    def _gather(table_hbm, idx_hbm, out_hbm, idx_v, row_v, rows_v, sem):
        wid = lax.axis_index("s") * _NUM_CORES + lax.axis_index("c")
        base = wid * b_per_w
        pltpu.sync_copy(idx_hbm.at[pl.ds(base, b_per_w)], idx_v)

        @pl.loop(0, b_per_w, step=_LANES)
        def _(i):
            sl = pl.ds(i, _LANES)
            row_v[sl] = lax.shift_right_logical(idx_v[sl], 2)

        pltpu.async_copy(table_hbm.at[row_v], rows_v, sem).wait()
        pltpu.sync_copy(rows_v, out_hbm.at[pl.ds(base, b_per_w)])

    return _gather(table2, idx32)


def _tc_extract(packed, idxcol):
    batch, width = packed.shape
    dim = width // _PACK
    block = 2048

    def _body(p_ref, i_ref, o_ref):
        ph = i_ref[...] & _PACK - 1
        acc = jnp.zeros((block, dim), jnp.float32)
        for k in range(_PACK):
            acc = acc + jnp.where(
                ph == k, p_ref[:, k * dim:(k + 1) * dim], 0.0
            )
        o_ref[...] = acc

    return pl.pallas_call(
        _body,
        grid=(batch // block,),
        in_specs=[
            pl.BlockSpec((block, width), lambda m: (m, 0)),
            pl.BlockSpec((block, 1), lambda m: (m, 0)),
        ],
        out_specs=pl.BlockSpec((block, dim), lambda m: (m, 0)),
        out_shape=jax.ShapeDtypeStruct((batch, dim), packed.dtype),
    )(packed, idxcol)


def kernel(idx, table):
    n, dim = table.shape
    idx32 = idx.astype(jnp.int32)
    table2 = table.reshape(n // _PACK, dim * _PACK)
    packed = _sc_gather_packed(idx32, table2)
    return _tc_extract(packed, idx32.reshape(-1, 1))


# speedup vs baseline: 3.9788x; 3.9788x over previous
import jax
import jax.numpy as jnp
from jax import lax
from jax.experimental import pallas as pl
from jax.experimental.pallas import tpu as pltpu
from jax.experimental.pallas import tpu_sc as plsc

_NUM_CORES = 2
_NUM_SUBCORES = 16
_NW = _NUM_CORES * _NUM_SUBCORES
_CH = 8


def _sc_col_gather(idx32, x_t):
    dim = x_t.shape[0]
    batch = idx32.shape[0]
    b_per_w = batch // _NW
    n_pairs = b_per_w // (2 * _CH)

    mesh = plsc.VectorSubcoreMesh(core_axis_name="c", subcore_axis_name="s")

    @pl.kernel(
        out_type=jax.ShapeDtypeStruct((dim, batch), x_t.dtype),
        mesh=mesh,
        scratch_types=[
            pltpu.VMEM((b_per_w,), jnp.int32),
            pltpu.VMEM((_CH, dim, 128), x_t.dtype),
            pltpu.VMEM((_CH, dim, 128), x_t.dtype),
            pltpu.VMEM((dim, b_per_w), x_t.dtype),
            pltpu.SemaphoreType.DMA,
            pltpu.SemaphoreType.DMA,
            pltpu.SemaphoreType.DMA,
        ],
        compiler_params=pltpu.CompilerParams(needs_layout_passes=False),
    )
    def _gather(x_hbm, idx_hbm, out_hbm, idx_v, buf0, buf1, cols, sem0, sem1,
                osem):
        wid = lax.axis_index("s") * _NUM_CORES + lax.axis_index("c")
        base = wid * b_per_w
        pltpu.async_copy(idx_hbm.at[pl.ds(base, b_per_w)], idx_v, osem).wait()
        i16 = jax.lax.iota(jnp.int32, 16)

        def issue_chunk(p, half, buf, sem):
            v = idx_v[pl.ds(p * 2 * _CH, 16)]
            for j in range(_CH):
                t = v[half * _CH + j]
                g = pl.multiple_of((t >> 7) << 7, 128)
                pltpu.async_copy(
                    x_hbm.at[:, pl.ds(g, 128)], buf.at[j], sem
                )

        def drain_extract_chunk(p, half, buf, sem):
            for j in range(_CH):
                pltpu.make_async_copy(
                    x_hbm.at[:, pl.ds(0, 128)], buf.at[j], sem
                ).wait()
            v = idx_v[pl.ds(p * 2 * _CH, 16)]
            for j in range(_CH):
                t = v[half * _CH + j]
                pv = jnp.full((16,), t & 127, jnp.int32)
                cv = jnp.full((16,), p * 2 * _CH + half * _CH + j, jnp.int32)
                v0 = plsc.load_gather(buf.at[j], [i16, pv])
                v1 = plsc.load_gather(buf.at[j], [i16 + 16, pv])
                plsc.store_scatter(cols, [i16, cv], v0)
                plsc.store_scatter(cols, [i16 + 16, cv], v1)

        issue_chunk(0, 0, buf0, sem0)

        @pl.loop(0, n_pairs)
        def _(p):
            issue_chunk(p, 1, buf1, sem1)
            drain_extract_chunk(p, 0, buf0, sem0)

            @pl.when(p + 1 < n_pairs)
            def _():
                issue_chunk(p + 1, 0, buf0, sem0)

            drain_extract_chunk(p, 1, buf1, sem1)

        pltpu.async_copy(
            cols, out_hbm.at[:, pl.ds(base, b_per_w)], osem
        ).wait()

    return _gather(x_t, idx32)


def kernel(idx, table):
    idx32 = idx.astype(jnp.int32)
    out_t = _sc_col_gather(idx32, table.T)
    return out_t.T
